# baseline (speedup 1.0000x reference)
import jax
import jax.numpy as jnp
from jax.experimental import pallas as pl

_BM = 1024


def _router_block(x_ref, w_ref, b_ref, o_ref):
    acc = jax.lax.dot_general(
        x_ref[...].astype(jnp.bfloat16),
        w_ref[...].astype(jnp.bfloat16),
        dimension_numbers=(((1,), (1,)), ((), ())),
        preferred_element_type=jnp.float32,
    )
    o_ref[...] = acc + b_ref[...]


def kernel(x, W, b):
    n_tokens, d_model = x.shape
    n_experts = W.shape[0]
    b2 = b.reshape(1, n_experts)
    return pl.pallas_call(
        _router_block,
        grid=(n_tokens // _BM,),
        in_specs=[
            pl.BlockSpec((_BM, d_model), lambda i: (i, 0)),
            pl.BlockSpec((n_experts, d_model), lambda i: (0, 0)),
            pl.BlockSpec((1, n_experts), lambda i: (0, 0)),
        ],
        out_specs=pl.BlockSpec((_BM, n_experts), lambda i: (i, 0)),
        out_shape=jax.ShapeDtypeStruct((n_tokens, n_experts), jnp.float32),
    )(x, W, b2)


# speedup vs baseline: 1.0385x; 1.0385x over previous
import jax
import jax.numpy as jnp
from jax.experimental import pallas as pl

_BM = 512


def _router_block(x_ref, w_ref, b_ref, o_ref):
    acc = jax.lax.dot_general(
        x_ref[...].astype(jnp.bfloat16),
        w_ref[...].astype(jnp.bfloat16),
        dimension_numbers=(((1,), (1,)), ((), ())),
        preferred_element_type=jnp.float32,
    )
    o_ref[...] = acc + b_ref[...]


def kernel(x, W, b):
    n_tokens, d_model = x.shape
    n_experts = W.shape[0]
    b2 = b.reshape(1, n_experts)
    return pl.pallas_call(
        _router_block,
        grid=(n_tokens // _BM,),
        in_specs=[
            pl.BlockSpec((_BM, d_model), lambda i: (i, 0)),
            pl.BlockSpec((n_experts, d_model), lambda i: (0, 0)),
            pl.BlockSpec((1, n_experts), lambda i: (0, 0)),
        ],
        out_specs=pl.BlockSpec((_BM, n_experts), lambda i: (i, 0)),
        out_shape=jax.ShapeDtypeStruct((n_tokens, n_experts), jnp.float32),
    )(x, W, b2)
